# baseline (speedup 1.0000x reference)
import jax
import jax.numpy as jnp
from jax import lax
from jax.experimental import pallas as pl
from jax.experimental.pallas import tpu as pltpu
from jax.experimental.pallas import tpu_sc as plsc

N = 10000
E = 320000
F = 128
R = 8
NC = 2
NS = 16
LANES = 16
FCN = F // LANES
FCPC = FCN // NC
NSLOT = N * R
DUMP = 128
B = 512
G = 128
NG = B // G
EPT = 20480
EPAD = EPT * NS
NBATCH = EPT // B
EPT_H = EPAD // (2 * NS)
NBATCH_H = EPT_H // B

NBLK = 1000


def _make_sc_agg(with_cnt):
  zb = NSLOT // NS // 2
  wpt = NSLOT // NS
  ngt = EPT // G
  mesh = plsc.VectorSubcoreMesh(
      core_axis_name="c", subcore_axis_name="s",
      num_cores=NC, num_subcores=NS)
  out_type = [jax.ShapeDtypeStruct((FCN, NSLOT, LANES), jnp.float32)]
  if with_cnt:
    out_type.append(jax.ShapeDtypeStruct((NC, NSLOT, LANES), jnp.float32))
  scratch = [
      pltpu.VMEM((B,), jnp.int32),
      pltpu.VMEM((B,), jnp.int32),
      pltpu.VMEM((B,), jnp.int32),
      pltpu.VMEM((ngt, G), jnp.int32),
      pltpu.VMEM((ngt, G), jnp.int32),
      pltpu.VMEM((B, LANES), jnp.float32),
      pltpu.VMEM_SHARED((NSLOT, LANES), jnp.float32),
      pltpu.SemaphoreType.DMA,
      pltpu.SemaphoreType.DMA,
  ]

  def body(xt, srcr, dstr, etr, zsrc, *rest):
    if with_cnt:
      (agg, cnt, src_v, dst_v, et_v, gbuf, sbuf, rows, acc, sem, sem2) = rest
    else:
      (agg, src_v, dst_v, et_v, gbuf, sbuf, rows, acc, sem, sem2) = rest
      cnt = None
    c = lax.axis_index("c")
    s = lax.axis_index("s")

    def zero_acc():
      base = s * wpt
      pltpu.sync_copy(zsrc, acc.at[pl.ds(base, zb)])
      pltpu.sync_copy(zsrc, acc.at[pl.ds(base + zb, zb)])

    def pre_body(pb, carry):
      ebase = s * EPT + pb * B
      ldescs = [
          pltpu.async_copy(srcr.at[pl.ds(ebase, B)], src_v, sem),
          pltpu.async_copy(dstr.at[pl.ds(ebase, B)], dst_v, sem),
          pltpu.async_copy(etr.at[pl.ds(ebase, B)], et_v, sem),
      ]
      for d in ldescs:
        d.wait()

      def pcb(i, carry3):
        gg = pb * NG + i // (G // LANES)
        col = (i % (G // LANES)) * LANES
        sv = src_v[pl.ds(i * LANES, LANES)]
        dv = dst_v[pl.ds(i * LANES, LANES)]
        ev = et_v[pl.ds(i * LANES, LANES)]
        slot = dv * R + ev
        oob = slot >= NSLOT
        sbuf[gg, pl.ds(col, LANES)] = jnp.where(oob, -1, slot)
        gbuf[gg, pl.ds(col, LANES)] = jnp.where(oob, -1, sv)
        return carry3
      lax.fori_loop(0, B // LANES, pcb, 0)
      return carry
    lax.fori_loop(0, NBATCH, pre_body, 0)

    def chunk_body(cc, carry):
      fc = c * FCPC + cc
      table = xt.at[pl.ds(fc * N, N)]
      plsc.subcore_barrier()
      zero_acc()
      plsc.subcore_barrier()

      half_g = NG // 2

      def fire_gathers(hb, gbase):
        return [
            pltpu.async_copy(
                table.at[plsc.Indices(gbuf.at[gbase + g], ignored_value=-1)],
                rows.at[pl.ds((hb * half_g + g) * G, G)], sem)
            for g in range(half_g)
        ]

      def scatter_half(hb, gbase):
        sdescs = []
        for g in range(half_g):
          sdescs.append(pltpu.async_copy(
              rows.at[pl.ds((hb * half_g + g) * G, G)],
              acc.at[plsc.Indices(sbuf.at[gbase + g], ignored_value=-1)],
              sem2, add=True))
        return sdescs

      def wait_all(descs):
        for d in descs:
          d.wait()

      nhb = NBATCH * 2
      wait_all(fire_gathers(0, 0))

      def batch_body(k, carry2):
        gdescs = fire_gathers((k + 1) % 2, (k + 1) * half_g)
        sdescs = scatter_half(k % 2, k * half_g)
        wait_all(sdescs)
        wait_all(gdescs)
        return carry2
      lax.fori_loop(0, nhb - 1, batch_body, 0)
      wait_all(scatter_half((nhb - 1) % 2, (nhb - 1) * half_g))

      plsc.subcore_barrier()
      pltpu.sync_copy(acc.at[pl.ds(s * wpt, wpt)],
                      agg.at[fc, pl.ds(s * wpt, wpt)])
      return carry
    lax.fori_loop(0, FCPC, chunk_body, 0)

    if with_cnt:
      plsc.subcore_barrier()
      zero_acc()

      ones = jnp.ones((LANES,), jnp.float32)
      def fill(i, carry2):
        rows[i, :] = ones
        return carry2
      lax.fori_loop(0, G, fill, 0)
      plsc.subcore_barrier()

      def cnt_body(b, carry2):
        gbase = b * NG
        sdescs = [
            pltpu.async_copy(
                rows.at[pl.ds(0, G)],
                acc.at[plsc.Indices(sbuf.at[gbase + g], ignored_value=-1)],
                sem2, add=True)
            for g in range(NG)
        ]
        for d in sdescs:
          d.wait()
        return carry2
      lax.fori_loop(0, NBATCH, cnt_body, 0)

      plsc.subcore_barrier()
      pltpu.sync_copy(acc.at[pl.ds(s * wpt, wpt)],
                      cnt.at[c, pl.ds(s * wpt, wpt)])

  return pl.kernel(body, out_type=tuple(out_type), mesh=mesh,
                   scratch_types=scratch,
                   compiler_params=pltpu.CompilerParams(
                       use_tc_tiling_on_sc=False))


_sc_agg_l1 = _make_sc_agg(True)
_sc_agg_l2 = _make_sc_agg(False)


def _tc_layer(x, agg, cnt, root, wt, bias, relu):
  def tc_body(x_r, agg_r, cnt_r, root_r, wt_r, b_r, o_r):
    acc = jnp.dot(x_r[...], root_r[...], preferred_element_type=jnp.float32)
    acc = acc + b_r[...]
    cs = cnt_r[...]
    scale = 1.0 / jnp.maximum((cs[0] + cs[1]) * 0.5, 1.0)
    ag = agg_r[...]
    for fc in range(FCN):
      acc = acc + jnp.dot(ag[fc] * scale, wt_r[fc],
                          preferred_element_type=jnp.float32)
    if relu:
      acc = jnp.maximum(acc, 0.0)
    o_r[...] = acc

  return pl.pallas_call(
      tc_body,
      grid=(N // NBLK,),
      in_specs=[
          pl.BlockSpec((NBLK, F), lambda j: (j, 0)),
          pl.BlockSpec((FCN, NBLK, F), lambda j: (0, j, 0)),
          pl.BlockSpec((NC, NBLK, F), lambda j: (0, j, 0)),
          pl.BlockSpec((F, F), lambda j: (0, 0)),
          pl.BlockSpec((FCN, F, F), lambda j: (0, 0, 0)),
          pl.BlockSpec((1, F), lambda j: (0, 0)),
      ],
      out_specs=pl.BlockSpec((NBLK, F), lambda j: (j, 0)),
      out_shape=jax.ShapeDtypeStruct((N, F), jnp.float32),
  )(x, agg, cnt, root, wt, bias)


def kernel(x, A, edge_type, W1, root1, b1, W2, root2, b2):
  src = A[0]
  dst = A[1]
  pad = EPAD - E
  k = jnp.arange(pad, dtype=jnp.int32)
  srcp = jnp.concatenate([src, k % N])
  dstp = jnp.concatenate([dst, N + (k % DUMP) // R])
  etp = jnp.concatenate([edge_type, k % R])

  wt1 = W1.reshape(R, FCN, LANES, F).transpose(1, 0, 2, 3).reshape(FCN, F, F)
  wt2 = W2.reshape(R, FCN, LANES, F).transpose(1, 0, 2, 3).reshape(FCN, F, F)

  zeros = jnp.zeros((NSLOT // NS // 2, LANES), jnp.float32)

  xt = x.reshape(N, FCN, LANES).transpose(1, 0, 2).reshape(FCN * N, LANES)
  agg1, cnt = _sc_agg_l1(xt, srcp, dstp, etp, zeros)
  aggv1 = agg1.reshape(FCN, N, F)
  cntv = cnt.reshape(NC, N, F)
  h = _tc_layer(x, aggv1, cntv, root1, wt1, b1.reshape(1, F), relu=True)

  ht = h.reshape(N, FCN, LANES).transpose(1, 0, 2).reshape(FCN * N, LANES)
  agg2 = _sc_agg_l2(ht, srcp, dstp, etp, zeros)
  if isinstance(agg2, (list, tuple)):
    agg2 = agg2[0]
  out = _tc_layer(h, agg2.reshape(FCN, N, F), cntv, root2, wt2,
                  b2.reshape(1, F), relu=False)
  return out


# speedup vs baseline: 1.1579x; 1.1579x over previous
import jax
import jax.numpy as jnp
from jax import lax
from jax.experimental import pallas as pl
from jax.experimental.pallas import tpu as pltpu
from jax.experimental.pallas import tpu_sc as plsc

N = 10000
E = 320000
F = 128
R = 8
NC = 2
NS = 16
LANES = 16
FCN = F // LANES
FCPC = FCN // NC
NSLOT = N * R
DUMP = 128
B = 512
G = 128
NG = B // G
EPT = 20480
EPAD = EPT * NS
NBATCH = EPT // B
EPT_H = EPAD // (2 * NS)
NBATCH_H = EPT_H // B

NBLK = 1000


def _make_sc_agg(with_cnt):
  zb = NSLOT // NS // 2
  wpt = NSLOT // NS
  ngt = EPT // G
  mesh = plsc.VectorSubcoreMesh(
      core_axis_name="c", subcore_axis_name="s",
      num_cores=NC, num_subcores=NS)
  out_type = [jax.ShapeDtypeStruct((FCN, NSLOT, LANES), jnp.float32)]
  if with_cnt:
    out_type.append(jax.ShapeDtypeStruct((NC, NSLOT, LANES), jnp.float32))
  scratch = [
      pltpu.VMEM((B,), jnp.int32),
      pltpu.VMEM((B,), jnp.int32),
      pltpu.VMEM((B,), jnp.int32),
      pltpu.VMEM((ngt, G), jnp.int32),
      pltpu.VMEM((ngt, G), jnp.int32),
      pltpu.VMEM((B, LANES), jnp.float32),
      pltpu.VMEM_SHARED((NSLOT, LANES), jnp.float32),
      pltpu.SemaphoreType.DMA,
      pltpu.SemaphoreType.DMA,
  ]

  def body(xt, srcr, dstr, etr, zsrc, *rest):
    if with_cnt:
      (agg, cnt, src_v, dst_v, et_v, gbuf, sbuf, rows, acc, sem, sem2) = rest
    else:
      (agg, src_v, dst_v, et_v, gbuf, sbuf, rows, acc, sem, sem2) = rest
      cnt = None
    c = lax.axis_index("c")
    s = lax.axis_index("s")

    def zero_acc():
      base = s * wpt
      pltpu.sync_copy(zsrc, acc.at[pl.ds(base, zb)])
      pltpu.sync_copy(zsrc, acc.at[pl.ds(base + zb, zb)])

    def pre_body(pb, carry):
      ebase = s * EPT + pb * B
      ldescs = [
          pltpu.async_copy(srcr.at[pl.ds(ebase, B)], src_v, sem),
          pltpu.async_copy(dstr.at[pl.ds(ebase, B)], dst_v, sem),
          pltpu.async_copy(etr.at[pl.ds(ebase, B)], et_v, sem),
      ]
      for d in ldescs:
        d.wait()

      def pcb(i, carry3):
        gg = pb * NG + i // (G // LANES)
        col = (i % (G // LANES)) * LANES
        sv = src_v[pl.ds(i * LANES, LANES)]
        dv = dst_v[pl.ds(i * LANES, LANES)]
        ev = et_v[pl.ds(i * LANES, LANES)]
        slot = dv * R + ev
        oob = slot >= NSLOT
        sbuf[gg, pl.ds(col, LANES)] = jnp.where(oob, -1, slot)
        gbuf[gg, pl.ds(col, LANES)] = jnp.where(oob, -1, sv)
        return carry3
      lax.fori_loop(0, B // LANES, pcb, 0)
      return carry
    lax.fori_loop(0, NBATCH, pre_body, 0)

    def chunk_body(cc, carry):
      fc = c * FCPC + cc
      table = xt.at[pl.ds(fc * N, N)]
      plsc.subcore_barrier()
      zero_acc()
      plsc.subcore_barrier()

      def batch_body(b, carry2):
        gbase = b * NG
        descs = [
            pltpu.async_copy(
                table.at[plsc.Indices(gbuf.at[gbase + g], ignored_value=-1)],
                rows.at[pl.ds(g * G, G)], sem)
            for g in range(NG)
        ]
        sdescs = []
        for g in range(NG):
          descs[g].wait()
          sdescs.append(pltpu.async_copy(
              rows.at[pl.ds(g * G, G)],
              acc.at[plsc.Indices(sbuf.at[gbase + g], ignored_value=-1)],
              sem2, add=True))
        for d in sdescs:
          d.wait()
        return carry2
      lax.fori_loop(0, NBATCH, batch_body, 0)

      plsc.subcore_barrier()
      pltpu.sync_copy(acc.at[pl.ds(s * wpt, wpt)],
                      agg.at[fc, pl.ds(s * wpt, wpt)])
      return carry
    lax.fori_loop(0, FCPC, chunk_body, 0)

    if with_cnt:
      plsc.subcore_barrier()
      zero_acc()

      ones = jnp.ones((LANES,), jnp.float32)
      def fill(i, carry2):
        rows[i, :] = ones
        return carry2
      lax.fori_loop(0, G, fill, 0)
      plsc.subcore_barrier()

      def cnt_body(b, carry2):
        gbase = b * NG
        sdescs = [
            pltpu.async_copy(
                rows.at[pl.ds(0, G)],
                acc.at[plsc.Indices(sbuf.at[gbase + g], ignored_value=-1)],
                sem2, add=True)
            for g in range(NG)
        ]
        for d in sdescs:
          d.wait()
        return carry2
      lax.fori_loop(0, NBATCH, cnt_body, 0)

      plsc.subcore_barrier()
      pltpu.sync_copy(acc.at[pl.ds(s * wpt, wpt)],
                      cnt.at[c, pl.ds(s * wpt, wpt)])

  return pl.kernel(body, out_type=tuple(out_type), mesh=mesh,
                   scratch_types=scratch,
                   compiler_params=pltpu.CompilerParams(
                       use_tc_tiling_on_sc=False))


_sc_agg_l1 = _make_sc_agg(True)
_sc_agg_l2 = _make_sc_agg(False)


def _tc_layer(x, agg, cnt, root, wt, bias, relu):
  def tc_body(x_r, agg_r, cnt_r, root_r, wt_r, b_r, o_r):
    acc = jnp.dot(x_r[...], root_r[...], preferred_element_type=jnp.float32)
    acc = acc + b_r[...]
    cs = cnt_r[...]
    scale = 1.0 / jnp.maximum((cs[0] + cs[1]) * 0.5, 1.0)
    ag = agg_r[...]
    for fc in range(FCN):
      acc = acc + jnp.dot(ag[fc] * scale, wt_r[fc],
                          preferred_element_type=jnp.float32)
    if relu:
      acc = jnp.maximum(acc, 0.0)
    o_r[...] = acc

  return pl.pallas_call(
      tc_body,
      grid=(N // NBLK,),
      in_specs=[
          pl.BlockSpec((NBLK, F), lambda j: (j, 0)),
          pl.BlockSpec((FCN, NBLK, F), lambda j: (0, j, 0)),
          pl.BlockSpec((NC, NBLK, F), lambda j: (0, j, 0)),
          pl.BlockSpec((F, F), lambda j: (0, 0)),
          pl.BlockSpec((FCN, F, F), lambda j: (0, 0, 0)),
          pl.BlockSpec((1, F), lambda j: (0, 0)),
      ],
      out_specs=pl.BlockSpec((NBLK, F), lambda j: (j, 0)),
      out_shape=jax.ShapeDtypeStruct((N, F), jnp.float32),
  )(x, agg, cnt, root, wt, bias)


def kernel(x, A, edge_type, W1, root1, b1, W2, root2, b2):
  src = A[0]
  dst = A[1]
  pad = EPAD - E
  k = jnp.arange(pad, dtype=jnp.int32)
  srcp = jnp.concatenate([src, k % N])
  dstp = jnp.concatenate([dst, N + (k % DUMP) // R])
  etp = jnp.concatenate([edge_type, k % R])

  wt1 = W1.reshape(R, FCN, LANES, F).transpose(1, 0, 2, 3).reshape(FCN, F, F)
  wt2 = W2.reshape(R, FCN, LANES, F).transpose(1, 0, 2, 3).reshape(FCN, F, F)

  zeros = jnp.zeros((NSLOT // NS // 2, LANES), jnp.float32)

  xt = x.reshape(N, FCN, LANES).transpose(1, 0, 2).reshape(FCN * N, LANES)
  agg1, cnt = _sc_agg_l1(xt, srcp, dstp, etp, zeros)
  aggv1 = agg1.reshape(FCN, N, F)
  cntv = cnt.reshape(NC, N, F)
  h = _tc_layer(x, aggv1, cntv, root1, wt1, b1.reshape(1, F), relu=True)

  ht = h.reshape(N, FCN, LANES).transpose(1, 0, 2).reshape(FCN * N, LANES)
  agg2 = _sc_agg_l2(ht, srcp, dstp, etp, zeros)
  if isinstance(agg2, (list, tuple)):
    agg2 = agg2[0]
  out = _tc_layer(h, agg2.reshape(FCN, N, F), cntv, root2, wt2,
                  b2.reshape(1, F), relu=False)
  return out


# speedup vs baseline: 1.1967x; 1.0335x over previous
import jax
import jax.numpy as jnp
from jax import lax
from jax.experimental import pallas as pl
from jax.experimental.pallas import tpu as pltpu
from jax.experimental.pallas import tpu_sc as plsc

N = 10000
E = 320000
F = 128
R = 8
NC = 2
NS = 16
LANES = 16
FCN = F // LANES
FCPC = FCN // NC
NSLOT = N * R
DUMP = 128
B = 512
G = 128
NG = B // G
EPT = 20480
EPAD = EPT * NS
NBATCH = EPT // B
EPT_H = EPAD // (2 * NS)
NBATCH_H = EPT_H // B

NBLK = 1000


def _make_sc_agg(with_cnt):
  zb = NSLOT // NS // 2
  wpt = NSLOT // NS
  ngt = EPT // G
  mesh = plsc.VectorSubcoreMesh(
      core_axis_name="c", subcore_axis_name="s",
      num_cores=NC, num_subcores=NS)
  out_type = [jax.ShapeDtypeStruct((FCN, NSLOT, LANES), jnp.float32)]
  if with_cnt:
    out_type.append(jax.ShapeDtypeStruct((NC, NSLOT, LANES), jnp.float32))
    out_type.append(jax.ShapeDtypeStruct((NS * ngt, G), jnp.int32))
    out_type.append(jax.ShapeDtypeStruct((NS * ngt, G), jnp.int32))
  scratch = [
      pltpu.VMEM((B,), jnp.int32),
      pltpu.VMEM((B,), jnp.int32),
      pltpu.VMEM((B,), jnp.int32),
      pltpu.VMEM((ngt, G), jnp.int32),
      pltpu.VMEM((ngt, G), jnp.int32),
      pltpu.VMEM((B, LANES), jnp.float32),
      pltpu.VMEM_SHARED((NSLOT, LANES), jnp.float32),
      pltpu.SemaphoreType.DMA,
      pltpu.SemaphoreType.DMA,
  ]

  def body(*args):
    if with_cnt:
      (xt, srcr, dstr, etr, zsrc, agg, cnt, gidx_out, sidx_out,
       src_v, dst_v, et_v, gbuf, sbuf, rows, acc, sem, sem2) = args
    else:
      (xt, gidx_hbm, sidx_hbm, zsrc, agg,
       src_v, dst_v, et_v, gbuf, sbuf, rows, acc, sem, sem2) = args
      cnt = None
    c = lax.axis_index("c")
    s = lax.axis_index("s")

    def zero_acc():
      base = s * wpt
      pltpu.sync_copy(zsrc, acc.at[pl.ds(base, zb)])
      pltpu.sync_copy(zsrc, acc.at[pl.ds(base + zb, zb)])

    if with_cnt:
      def pre_body(pb, carry):
        ebase = s * EPT + pb * B
        ldescs = [
            pltpu.async_copy(srcr.at[pl.ds(ebase, B)], src_v, sem),
            pltpu.async_copy(dstr.at[pl.ds(ebase, B)], dst_v, sem),
            pltpu.async_copy(etr.at[pl.ds(ebase, B)], et_v, sem),
        ]
        for d in ldescs:
          d.wait()

        def pcb(i, carry3):
          gg = pb * NG + i // (G // LANES)
          col = (i % (G // LANES)) * LANES
          sv = src_v[pl.ds(i * LANES, LANES)]
          dv = dst_v[pl.ds(i * LANES, LANES)]
          ev = et_v[pl.ds(i * LANES, LANES)]
          slot = dv * R + ev
          oob = slot >= NSLOT
          sbuf[gg, pl.ds(col, LANES)] = jnp.where(oob, -1, slot)
          gbuf[gg, pl.ds(col, LANES)] = jnp.where(oob, -1, sv)
          return carry3
        lax.fori_loop(0, B // LANES, pcb, 0)
        return carry
      lax.fori_loop(0, NBATCH, pre_body, 0)

      @pl.when(c == 0)
      def _():
        pltpu.sync_copy(gbuf, gidx_out.at[pl.ds(s * ngt, ngt)])
        pltpu.sync_copy(sbuf, sidx_out.at[pl.ds(s * ngt, ngt)])
    else:
      pltpu.sync_copy(gidx_hbm.at[pl.ds(s * ngt, ngt)], gbuf)
      pltpu.sync_copy(sidx_hbm.at[pl.ds(s * ngt, ngt)], sbuf)

    def chunk_body(cc, carry):
      fc = c * FCPC + cc
      table = xt.at[pl.ds(fc * N, N)]
      plsc.subcore_barrier()
      zero_acc()
      plsc.subcore_barrier()

      def batch_body(b, carry2):
        gbase = b * NG
        descs = [
            pltpu.async_copy(
                table.at[plsc.Indices(gbuf.at[gbase + g], ignored_value=-1)],
                rows.at[pl.ds(g * G, G)], sem)
            for g in range(NG)
        ]
        sdescs = []
        for g in range(NG):
          descs[g].wait()
          sdescs.append(pltpu.async_copy(
              rows.at[pl.ds(g * G, G)],
              acc.at[plsc.Indices(sbuf.at[gbase + g], ignored_value=-1)],
              sem2, add=True))
        for d in sdescs:
          d.wait()
        return carry2
      lax.fori_loop(0, NBATCH, batch_body, 0)

      plsc.subcore_barrier()
      pltpu.sync_copy(acc.at[pl.ds(s * wpt, wpt)],
                      agg.at[fc, pl.ds(s * wpt, wpt)])
      return carry
    lax.fori_loop(0, FCPC, chunk_body, 0)

    if with_cnt:
      plsc.subcore_barrier()
      zero_acc()

      ones = jnp.ones((LANES,), jnp.float32)
      def fill(i, carry2):
        rows[i, :] = ones
        return carry2
      lax.fori_loop(0, G, fill, 0)
      plsc.subcore_barrier()

      def cnt_body(b, carry2):
        gbase = b * NG
        sdescs = [
            pltpu.async_copy(
                rows.at[pl.ds(0, G)],
                acc.at[plsc.Indices(sbuf.at[gbase + g], ignored_value=-1)],
                sem2, add=True)
            for g in range(NG)
        ]
        for d in sdescs:
          d.wait()
        return carry2
      lax.fori_loop(0, NBATCH, cnt_body, 0)

      plsc.subcore_barrier()
      pltpu.sync_copy(acc.at[pl.ds(s * wpt, wpt)],
                      cnt.at[c, pl.ds(s * wpt, wpt)])

  return pl.kernel(body, out_type=tuple(out_type), mesh=mesh,
                   scratch_types=scratch,
                   compiler_params=pltpu.CompilerParams(
                       use_tc_tiling_on_sc=False))


_sc_agg_l1 = _make_sc_agg(True)
_sc_agg_l2 = _make_sc_agg(False)


def _tc_layer(x, agg, cnt, root, wt, bias, relu):
  def tc_body(x_r, agg_r, cnt_r, root_r, wt_r, b_r, o_r):
    acc = jnp.dot(x_r[...], root_r[...], preferred_element_type=jnp.float32)
    acc = acc + b_r[...]
    cs = cnt_r[...]
    scale = 1.0 / jnp.maximum((cs[0] + cs[1]) * 0.5, 1.0)
    ag = agg_r[...]
    for fc in range(FCN):
      acc = acc + jnp.dot(ag[fc] * scale, wt_r[fc],
                          preferred_element_type=jnp.float32)
    if relu:
      acc = jnp.maximum(acc, 0.0)
    o_r[...] = acc

  return pl.pallas_call(
      tc_body,
      grid=(N // NBLK,),
      in_specs=[
          pl.BlockSpec((NBLK, F), lambda j: (j, 0)),
          pl.BlockSpec((FCN, NBLK, F), lambda j: (0, j, 0)),
          pl.BlockSpec((NC, NBLK, F), lambda j: (0, j, 0)),
          pl.BlockSpec((F, F), lambda j: (0, 0)),
          pl.BlockSpec((FCN, F, F), lambda j: (0, 0, 0)),
          pl.BlockSpec((1, F), lambda j: (0, 0)),
      ],
      out_specs=pl.BlockSpec((NBLK, F), lambda j: (j, 0)),
      out_shape=jax.ShapeDtypeStruct((N, F), jnp.float32),
  )(x, agg, cnt, root, wt, bias)


def kernel(x, A, edge_type, W1, root1, b1, W2, root2, b2):
  src = A[0]
  dst = A[1]
  pad = EPAD - E
  k = jnp.arange(pad, dtype=jnp.int32)
  srcp = jnp.concatenate([src, k % N])
  dstp = jnp.concatenate([dst, N + (k % DUMP) // R])
  etp = jnp.concatenate([edge_type, k % R])

  wt1 = W1.reshape(R, FCN, LANES, F).transpose(1, 0, 2, 3).reshape(FCN, F, F)
  wt2 = W2.reshape(R, FCN, LANES, F).transpose(1, 0, 2, 3).reshape(FCN, F, F)

  zeros = jnp.zeros((NSLOT // NS // 2, LANES), jnp.float32)

  xt = x.reshape(N, FCN, LANES).transpose(1, 0, 2).reshape(FCN * N, LANES)
  agg1, cnt, gidxs, sidxs = _sc_agg_l1(xt, srcp, dstp, etp, zeros)
  aggv1 = agg1.reshape(FCN, N, F)
  cntv = cnt.reshape(NC, N, F)
  h = _tc_layer(x, aggv1, cntv, root1, wt1, b1.reshape(1, F), relu=True)

  ht = h.reshape(N, FCN, LANES).transpose(1, 0, 2).reshape(FCN * N, LANES)
  agg2 = _sc_agg_l2(ht, gidxs, sidxs, zeros)
  if isinstance(agg2, (list, tuple)):
    agg2 = agg2[0]
  out = _tc_layer(h, agg2.reshape(FCN, N, F), cntv, root2, wt2,
                  b2.reshape(1, F), relu=False)
  return out


# speedup vs baseline: 1.1970x; 1.0003x over previous
import jax
import jax.numpy as jnp
from jax import lax
from jax.experimental import pallas as pl
from jax.experimental.pallas import tpu as pltpu
from jax.experimental.pallas import tpu_sc as plsc

N = 10000
E = 320000
F = 128
R = 8
NC = 2
NS = 16
LANES = 16
FCN = F // LANES
FCPC = FCN // NC
NSLOT = N * R
DUMP = 128
B = 512
G = 128
NG = B // G
EPT = 20480
EPAD = EPT * NS
NBATCH = EPT // B

NBLK = 1000


def _make_sc_agg(with_cnt):
  zb = NSLOT // NS // 2
  wpt = NSLOT // NS
  ngt = EPT // G
  mesh = plsc.VectorSubcoreMesh(
      core_axis_name="c", subcore_axis_name="s",
      num_cores=NC, num_subcores=NS)
  out_type = [jax.ShapeDtypeStruct((FCN, NSLOT, LANES), jnp.float32)]
  if with_cnt:
    out_type.append(jax.ShapeDtypeStruct((NC, NSLOT, LANES), jnp.float32))
    out_type.append(jax.ShapeDtypeStruct((NS * ngt, G), jnp.int32))
    out_type.append(jax.ShapeDtypeStruct((NS * ngt, G), jnp.int32))
  scratch = [
      pltpu.VMEM((B,), jnp.int32),
      pltpu.VMEM((B,), jnp.int32),
      pltpu.VMEM((B,), jnp.int32),
      pltpu.VMEM((ngt, G), jnp.int32),
      pltpu.VMEM((ngt, G), jnp.int32),
      pltpu.VMEM((B, LANES), jnp.float32),
      pltpu.VMEM_SHARED((NSLOT, LANES), jnp.float32),
      pltpu.SemaphoreType.DMA,
      pltpu.SemaphoreType.DMA,
  ]

  def body(*args):
    if with_cnt:
      (xt, srcr, dstr, etr, zsrc, agg, cnt, gidx_out, sidx_out,
       src_v, dst_v, et_v, gbuf, sbuf, rows, acc, sem, sem2) = args
    else:
      (xt, gidx_hbm, sidx_hbm, zsrc, agg,
       src_v, dst_v, et_v, gbuf, sbuf, rows, acc, sem, sem2) = args
      cnt = None
    c = lax.axis_index("c")
    s = lax.axis_index("s")

    def zero_acc():
      base = s * wpt
      pltpu.sync_copy(zsrc, acc.at[pl.ds(base, zb)])
      pltpu.sync_copy(zsrc, acc.at[pl.ds(base + zb, zb)])

    if with_cnt:
      def pre_body(pb, carry):
        ebase = s * EPT + pb * B
        ldescs = [
            pltpu.async_copy(srcr.at[pl.ds(ebase, B)], src_v, sem),
            pltpu.async_copy(dstr.at[pl.ds(ebase, B)], dst_v, sem),
            pltpu.async_copy(etr.at[pl.ds(ebase, B)], et_v, sem),
        ]
        for d in ldescs:
          d.wait()

        def pcb(i, carry3):
          gg = pb * NG + i // (G // LANES)
          col = (i % (G // LANES)) * LANES
          sv = src_v[pl.ds(i * LANES, LANES)]
          dv = dst_v[pl.ds(i * LANES, LANES)]
          ev = et_v[pl.ds(i * LANES, LANES)]
          slot = dv * R + ev
          oob = slot >= NSLOT
          sbuf[gg, pl.ds(col, LANES)] = jnp.where(oob, -1, slot)
          gbuf[gg, pl.ds(col, LANES)] = jnp.where(oob, -1, sv)
          return carry3
        lax.fori_loop(0, B // LANES, pcb, 0)
        return carry
      lax.fori_loop(0, NBATCH, pre_body, 0)

      @pl.when(c == 0)
      def _():
        pltpu.sync_copy(gbuf, gidx_out.at[pl.ds(s * ngt, ngt)])
        pltpu.sync_copy(sbuf, sidx_out.at[pl.ds(s * ngt, ngt)])
    else:
      pltpu.sync_copy(gidx_hbm.at[pl.ds(s * ngt, ngt)], gbuf)
      pltpu.sync_copy(sidx_hbm.at[pl.ds(s * ngt, ngt)], sbuf)

    def chunk_body(cc, carry):
      fc = c * FCPC + cc
      table = xt.at[pl.ds(fc * N, N)]
      plsc.subcore_barrier()
      zero_acc()
      plsc.subcore_barrier()

      def batch_body(b, carry2):
        gbase = b * NG
        descs = [
            pltpu.async_copy(
                table.at[plsc.Indices(gbuf.at[gbase + g], ignored_value=-1)],
                rows.at[pl.ds(g * G, G)], sem)
            for g in range(NG)
        ]
        sdescs = []
        for g in range(NG):
          descs[g].wait()
          sdescs.append(pltpu.async_copy(
              rows.at[pl.ds(g * G, G)],
              acc.at[plsc.Indices(sbuf.at[gbase + g], ignored_value=-1)],
              sem2, add=True))
        for d in sdescs:
          d.wait()
        return carry2
      lax.fori_loop(0, NBATCH, batch_body, 0)

      plsc.subcore_barrier()
      pltpu.sync_copy(acc.at[pl.ds(s * wpt, wpt)],
                      agg.at[fc, pl.ds(s * wpt, wpt)])
      return carry
    lax.fori_loop(0, FCPC, chunk_body, 0)

    if with_cnt:
      plsc.subcore_barrier()
      zero_acc()

      ones = jnp.ones((LANES,), jnp.float32)
      def fill(i, carry2):
        rows[i, :] = ones
        return carry2
      lax.fori_loop(0, G, fill, 0)
      plsc.subcore_barrier()

      def cnt_body(b, carry2):
        gbase = b * NG
        sdescs = [
            pltpu.async_copy(
                rows.at[pl.ds(0, G)],
                acc.at[plsc.Indices(sbuf.at[gbase + g], ignored_value=-1)],
                sem2, add=True)
            for g in range(NG)
        ]
        for d in sdescs:
          d.wait()
        return carry2
      lax.fori_loop(0, NBATCH, cnt_body, 0)

      plsc.subcore_barrier()
      pltpu.sync_copy(acc.at[pl.ds(s * wpt, wpt)],
                      cnt.at[c, pl.ds(s * wpt, wpt)])

  return pl.kernel(body, out_type=tuple(out_type), mesh=mesh,
                   scratch_types=scratch,
                   compiler_params=pltpu.CompilerParams(
                       use_tc_tiling_on_sc=False))


_sc_agg_l1 = _make_sc_agg(True)
_sc_agg_l2 = _make_sc_agg(False)


def _tc_layer(x, agg, cnt, root, wt, bias, relu):
  def tc_body(x_r, agg_r, cnt_r, root_r, wt_r, b_r, o_r):
    acc = jnp.dot(x_r[...], root_r[...], preferred_element_type=jnp.float32)
    acc = acc + b_r[...]
    cs = cnt_r[...]
    scale = 1.0 / jnp.maximum((cs[0] + cs[1]) * 0.5, 1.0)
    ag = agg_r[...]
    for fc in range(FCN):
      acc = acc + jnp.dot(ag[fc] * scale, wt_r[fc],
                          preferred_element_type=jnp.float32)
    if relu:
      acc = jnp.maximum(acc, 0.0)
    o_r[...] = acc

  return pl.pallas_call(
      tc_body,
      grid=(N // NBLK,),
      in_specs=[
          pl.BlockSpec((NBLK, F), lambda j: (j, 0)),
          pl.BlockSpec((FCN, NBLK, F), lambda j: (0, j, 0)),
          pl.BlockSpec((NC, NBLK, F), lambda j: (0, j, 0)),
          pl.BlockSpec((F, F), lambda j: (0, 0)),
          pl.BlockSpec((FCN, F, F), lambda j: (0, 0, 0)),
          pl.BlockSpec((1, F), lambda j: (0, 0)),
      ],
      out_specs=pl.BlockSpec((NBLK, F), lambda j: (j, 0)),
      out_shape=jax.ShapeDtypeStruct((N, F), jnp.float32),
  )(x, agg, cnt, root, wt, bias)


def kernel(x, A, edge_type, W1, root1, b1, W2, root2, b2):
  src = A[0]
  dst = A[1]
  pad = EPAD - E
  k = jnp.arange(pad, dtype=jnp.int32)
  srcp = jnp.concatenate([src, k % N])
  dstp = jnp.concatenate([dst, N + (k % DUMP) // R])
  etp = jnp.concatenate([edge_type, k % R])

  wt1 = W1.reshape(R, FCN, LANES, F).transpose(1, 0, 2, 3).reshape(FCN, F, F)
  wt2 = W2.reshape(R, FCN, LANES, F).transpose(1, 0, 2, 3).reshape(FCN, F, F)

  zeros = jnp.zeros((NSLOT // NS // 2, LANES), jnp.float32)

  xt = x.reshape(N, FCN, LANES).transpose(1, 0, 2).reshape(FCN * N, LANES)
  agg1, cnt, gidxs, sidxs = _sc_agg_l1(xt, srcp, dstp, etp, zeros)
  aggv1 = agg1.reshape(FCN, N, F)
  cntv = cnt.reshape(NC, N, F)
  h = _tc_layer(x, aggv1, cntv, root1, wt1, b1.reshape(1, F), relu=True)

  ht = h.reshape(N, FCN, LANES).transpose(1, 0, 2).reshape(FCN * N, LANES)
  agg2 = _sc_agg_l2(ht, gidxs, sidxs, zeros)
  if isinstance(agg2, (list, tuple)):
    agg2 = agg2[0]
  out = _tc_layer(h, agg2.reshape(FCN, N, F), cntv, root2, wt2,
                  b2.reshape(1, F), relu=False)
  return out
